# baseline (speedup 1.0000x reference)
import functools
import jax
import jax.numpy as jnp
from jax import lax
from jax.experimental import pallas as pl
from jax.experimental.pallas import tpu as pltpu, tpu_sc as plsc

_NC = 2
_NS = 16
_NW = _NC * _NS
_LANES = 16


def _policy_lr_sc(rows, cols, Lt, R, log_sigma):
    B = rows.shape[0]
    K, _ = Lt.shape
    assert B % _NW == 0 and K == 2 * _LANES
    NB = B // _NW
    NCH = NB // 128
    NBC = NB // _LANES

    mesh = plsc.VectorSubcoreMesh(
        core_axis_name="c", subcore_axis_name="s",
        num_cores=_NC, num_subcores=_NS)

    @functools.partial(
        pl.kernel,
        out_type=(jax.ShapeDtypeStruct((B,), jnp.float32),
                  jax.ShapeDtypeStruct((_LANES,), jnp.float32)),
        mesh=mesh,
        compiler_params=pltpu.CompilerParams(
            needs_layout_passes=False, use_tc_tiling_on_sc=False),
        scratch_types=[
            pltpu.VMEM((NCH, 128), jnp.int32),
            pltpu.VMEM((NCH, 128), jnp.int32),
            pltpu.VMEM((K * NB,), jnp.float32),
            pltpu.VMEM((K * NB,), jnp.float32),
            pltpu.VMEM((NB,), jnp.float32),
            pltpu.VMEM((_LANES,), jnp.float32),
            pltpu.SemaphoreType.DMA,
            pltpu.SemaphoreType.DMA,
        ],
    )
    def k(rows_h, cols_h, lt_h, r_h, sig_h, out_h, out2_h,
          rows_v, cols_v, lt_v, r_v, res_v, sig_v, lsem, rsem):
        wid = lax.axis_index("s") * _NC + lax.axis_index("c")
        base = wid * NB

        for i in range(NCH):
            pltpu.sync_copy(rows_h.at[pl.ds(base + i * 128, 128)],
                            rows_v.at[i])
            pltpu.sync_copy(cols_h.at[pl.ds(base + i * 128, 128)],
                            cols_v.at[i])

        copies = []
        for kk in range(K):
            for i in range(NCH):
                copies.append(pltpu.async_copy(
                    lt_h.at[kk].at[rows_v.at[i]],
                    lt_v.at[pl.ds(kk * NB + i * 128, 128)], lsem))
                copies.append(pltpu.async_copy(
                    r_h.at[kk].at[cols_v.at[i]],
                    r_v.at[pl.ds(kk * NB + i * 128, 128)], rsem))
        for cp in copies:
            cp.wait()

        def comp(bc, _):
            o = bc * _LANES
            acc = lt_v[pl.ds(o, _LANES)] * r_v[pl.ds(o, _LANES)]
            for kk in range(1, K):
                acc += (lt_v[pl.ds(kk * NB + o, _LANES)]
                        * r_v[pl.ds(kk * NB + o, _LANES)])
            res_v[pl.ds(o, _LANES)] = acc
            return 0

        lax.fori_loop(0, NBC, comp, 0)

        pltpu.sync_copy(res_v, out_h.at[pl.ds(base, NB)])

        @pl.when(wid == 0)
        def _():
            pltpu.sync_copy(sig_h, sig_v.at[pl.ds(0, 1)])
            v = sig_v[...]
            sig_v[...] = jnp.minimum(jnp.maximum(v, -2.5), 0.0)
            pltpu.sync_copy(sig_v, out2_h)

    return k(rows, cols, Lt, R, log_sigma)


def kernel(rows, cols, L, R, log_sigma):
    res, sig16 = _policy_lr_sc(rows.astype(jnp.int32), cols.astype(jnp.int32),
                               L.T, R, log_sigma)
    return res, sig16[:1]


# speedup vs baseline: 5.7076x; 5.7076x over previous
import functools
import jax
import jax.numpy as jnp
from jax import lax
from jax.experimental import pallas as pl
from jax.experimental.pallas import tpu as pltpu, tpu_sc as plsc

_NC = 2
_NS = 16
_NW = _NC * _NS
_LANES = 16


def _policy_lr_sc(rows, cols, L, Rt, log_sigma):
    B = rows.shape[0]
    K = L.shape[1]
    assert B % _NW == 0 and K == 2 * _LANES
    NB = B // _NW
    NCH = NB // 128

    mesh = plsc.VectorSubcoreMesh(
        core_axis_name="c", subcore_axis_name="s",
        num_cores=_NC, num_subcores=_NS)

    @functools.partial(
        pl.kernel,
        out_type=(jax.ShapeDtypeStruct((B,), jnp.float32),
                  jax.ShapeDtypeStruct((_LANES,), jnp.float32)),
        mesh=mesh,
        compiler_params=pltpu.CompilerParams(
            needs_layout_passes=False, use_tc_tiling_on_sc=False),
        scratch_types=[
            pltpu.VMEM((NCH, 128), jnp.int32),
            pltpu.VMEM((NCH, 128), jnp.int32),
            pltpu.VMEM((NB, K), jnp.float32),
            pltpu.VMEM((NB, K), jnp.float32),
            pltpu.VMEM((NB,), jnp.float32),
            pltpu.VMEM((_LANES,), jnp.float32),
            pltpu.SemaphoreType.DMA,
            pltpu.SemaphoreType.DMA,
        ],
    )
    def k(rows_h, cols_h, l_h, rt_h, sig_h, out_h, out2_h,
          rows_v, cols_v, l_v, rt_v, res_v, sig_v, lsem, rsem):
        wid = lax.axis_index("s") * _NC + lax.axis_index("c")
        base = wid * NB

        for i in range(NCH):
            pltpu.sync_copy(rows_h.at[pl.ds(base + i * 128, 128)],
                            rows_v.at[i])
            pltpu.sync_copy(cols_h.at[pl.ds(base + i * 128, 128)],
                            cols_v.at[i])

        copies = []
        for i in range(NCH):
            copies.append(pltpu.async_copy(
                l_h.at[rows_v.at[i]], l_v.at[pl.ds(i * 128, 128)], lsem))
            copies.append(pltpu.async_copy(
                rt_h.at[cols_v.at[i]], rt_v.at[pl.ds(i * 128, 128)], rsem))
        for cp in copies:
            cp.wait()

        lane = lax.iota(jnp.int32, _LANES)
        last = lane == (_LANES - 1)

        def comp(b, _):
            l0 = l_v[b, pl.ds(0, _LANES)]
            l1 = l_v[b, pl.ds(_LANES, _LANES)]
            r0 = rt_v[b, pl.ds(0, _LANES)]
            r1 = rt_v[b, pl.ds(_LANES, _LANES)]
            v = l0 * r0 + l1 * r1
            cs = plsc.cumsum(v)
            plsc.store_scatter(
                res_v, [jnp.full((_LANES,), b, jnp.int32)], cs, mask=last)
            return 0

        lax.fori_loop(0, NB, comp, 0)

        pltpu.sync_copy(res_v, out_h.at[pl.ds(base, NB)])

        @pl.when(wid == 0)
        def _():
            pltpu.sync_copy(sig_h, sig_v.at[pl.ds(0, 1)])
            v = sig_v[...]
            sig_v[...] = jnp.minimum(jnp.maximum(v, -2.5), 0.0)
            pltpu.sync_copy(sig_v, out2_h)

    return k(rows, cols, L, Rt, log_sigma)


def kernel(rows, cols, L, R, log_sigma):
    res, sig16 = _policy_lr_sc(rows.astype(jnp.int32), cols.astype(jnp.int32),
                               L, R.T, log_sigma)
    return res, sig16[:1]
